# baseline (speedup 1.0000x reference)
import functools

import jax
import jax.numpy as jnp
from jax import lax
from jax.experimental import pallas as pl
from jax.experimental.pallas import tpu as pltpu
from jax.experimental.pallas import tpu_sc as plsc


_TM = 512
_R = 1024
_QC = 1024
_RB = 8
_NQ = 4
_NG = 8
_ROWS_PG = _R // _NG
_NBATCH = _ROWS_PG // _RB


def _sc_rows_body(x_hbm, s_hbm, part_hbm, xq, wbuf, obuf, sem):
    cid = lax.axis_index("c")
    sid = lax.axis_index("s")
    wid = sid * 2 + cid
    q = wid % _NQ
    g = wid // _NQ
    qoff = q * _QC
    row0 = g * _ROWS_PG

    pltpu.sync_copy(x_hbm.at[pl.ds(q * (_QC // 2), _QC // 2), :], xq)

    def wcopy(batch, slot):
        return pltpu.make_async_copy(
            s_hbm.at[pl.ds(row0 + batch * _RB, _RB), pl.ds(qoff, _QC)],
            wbuf.at[slot],
            sem.at[slot],
        )

    wcopy(0, 0).start()

    @pl.loop(0, _NBATCH, step=2)
    def _batches(k):
        for u in range(2):
            batch = k + u

            @pl.when(batch + 1 < _NBATCH)
            def _():
                wcopy(batch + 1, 1 - u).start()

            wcopy(batch, u).wait()

            def block(cb, acc):
                acc = list(acc)
                base = cb * 16
                wv = [wbuf[u, b, pl.ds(base, 16)] for b in range(_RB)]
                for t in range(16):
                    xrow = cb * 8 + t // 2
                    xcol = (t % 2) * 64
                    xv = [xq[xrow, pl.ds(xcol + 16 * j, 16)] for j in range(4)]
                    idx = jnp.full((16,), t, jnp.int32)
                    for b in range(_RB):
                        wb = jnp.take_along_axis(
                            wv[b], idx, axis=0, mode="promise_in_bounds")
                        for j in range(4):
                            acc[b * 4 + j] = acc[b * 4 + j] + wb * xv[j]
                return tuple(acc)

            zero = jnp.zeros((16,), jnp.float32)
            acc = lax.fori_loop(0, _QC // 16, block, (zero,) * (_RB * 4))
            for b in range(_RB):
                for j in range(4):
                    obuf[b, 16 * j:16 * (j + 1)] = acc[b * 4 + j]
            pltpu.sync_copy(
                obuf, part_hbm.at[q, pl.ds(row0 + batch * _RB, _RB), :])


def _sc_rows(x_coarse, interp_matrix):
    kern = pl.kernel(
        _sc_rows_body,
        out_type=jax.ShapeDtypeStruct((_NQ, _R, 64), jnp.float32),
        mesh=plsc.VectorSubcoreMesh(core_axis_name="c", subcore_axis_name="s"),
        scratch_types=[
            pltpu.VMEM((_QC // 2, 128), jnp.float32),
            pltpu.VMEM((2, _RB, _QC), jnp.float32),
            pltpu.VMEM((_RB, 64), jnp.float32),
            pltpu.SemaphoreType.DMA((2,)),
        ],
    )
    return kern(x_coarse.reshape(_QC * _NQ // 2, 128), interp_matrix)


def _interp_tile(s_ref, x_ref, o_ref):
    o_ref[...] = jnp.dot(s_ref[...], x_ref[...],
                         preferred_element_type=jnp.float32)


def _tc_rows(x_coarse, interp_matrix):
    m, k = interp_matrix.shape
    n = x_coarse.shape[1]
    off = _R // _TM
    return pl.pallas_call(
        _interp_tile,
        grid=((m - _R) // _TM,),
        in_specs=[
            pl.BlockSpec((_TM, k), lambda i: (i + off, 0)),
            pl.BlockSpec(memory_space=pltpu.MemorySpace.VMEM),
        ],
        out_specs=pl.BlockSpec((_TM, n), lambda i: (i, 0)),
        out_shape=jax.ShapeDtypeStruct((m - _R, n), jnp.float32),
    )(interp_matrix, x_coarse)


def kernel(x_coarse, interp_matrix):
    part = _sc_rows(x_coarse, interp_matrix)
    rest = _tc_rows(x_coarse, interp_matrix)
    top = part[0] + part[1] + part[2] + part[3]
    return jnp.concatenate([top, rest], axis=0)


# speedup vs baseline: 3.0089x; 3.0089x over previous
import jax
import jax.numpy as jnp
from jax.experimental import pallas as pl
from jax.experimental.pallas import tpu as pltpu


_TM = 512


def _interp_tile(s_ref, x_ref, o_ref):
    o_ref[...] = jnp.dot(s_ref[...], x_ref[...],
                         preferred_element_type=jnp.float32)


def kernel(x_coarse, interp_matrix):
    m, k = interp_matrix.shape
    n = x_coarse.shape[1]
    return pl.pallas_call(
        _interp_tile,
        grid=(m // _TM,),
        in_specs=[
            pl.BlockSpec((_TM, k), lambda i: (i, 0)),
            pl.BlockSpec(memory_space=pltpu.MemorySpace.VMEM),
        ],
        out_specs=pl.BlockSpec((_TM, n), lambda i: (i, 0)),
        out_shape=jax.ShapeDtypeStruct((m, n), jnp.float32),
    )(interp_matrix, x_coarse)
